# baseline (speedup 1.0000x reference)
import functools

import jax
import jax.numpy as jnp
from jax import lax
from jax.experimental import pallas as pl
from jax.experimental.pallas import tpu as pltpu
from jax.experimental.pallas import tpu_sc as plsc

N = 10000
E = 320000
F = 128
F_OUT = 512
NGR = 64
K = 5

NC, NS = 2, 16
NW = NC * NS
CHUNK = 80
CPW = 256
EPW = CHUNK * CPW
EPAD = EPW * NS
NPAD = 10112
RPT = NPAD // NS

_mesh = plsc.VectorSubcoreMesh(core_axis_name="c", subcore_axis_name="s")
NBUF = 4


def _make_spmm(width):
    @functools.partial(
        pl.kernel,
        out_type=jax.ShapeDtypeStruct((NC, NPAD, width), jnp.float32),
        mesh=_mesh,
        scratch_types=[
            pltpu.VMEM((32, CHUNK), jnp.int32),
            pltpu.VMEM((32, CHUNK), jnp.int32),
            pltpu.VMEM_SHARED((NPAD, width), jnp.float32),
        ] + [pltpu.VMEM((CHUNK, width), jnp.float32) for _ in range(NBUF)]
          + [pltpu.SemaphoreType.DMA for _ in range(NBUF)],
    )
    def _spmm(u_hbm, src_hbm, dst_hbm, zero_hbm, out_hbm, sidx, didx, acc, *bufsems):
        rows = bufsems[:NBUF]
        sems = bufsems[NBUF:]
        cid = lax.axis_index("c")
        sid = lax.axis_index("s")
        ebase = (cid * NS + sid) * CPW
        r0 = sid * RPT
        pltpu.sync_copy(zero_hbm.at[pl.ds(r0, RPT)], acc.at[pl.ds(r0, RPT)])
        pltpu.sync_copy(src_hbm.at[pl.ds(ebase, 32)], sidx)
        pltpu.sync_copy(dst_hbm.at[pl.ds(ebase, 32)], didx)
        plsc.subcore_barrier()

        uplane = u_hbm.at[cid]

        def gather(c, b):
            pltpu.async_copy(uplane.at[sidx.at[lax.rem(c, 32)]], rows[b], sems[b])

        def wait_gather(c, b):
            pltpu.make_async_copy(uplane.at[sidx.at[lax.rem(c, 32)]],
                                  rows[b], sems[b]).wait()

        def scatter(c, b):
            pltpu.async_copy(rows[b], acc.at[didx.at[lax.rem(c, 32)]],
                             sems[b], add=True)

        def wait_scatter(c, b):
            pltpu.make_async_copy(rows[b], acc.at[didx.at[lax.rem(c, 32)]],
                                  sems[b]).wait()

        gather(0, 0)
        gather(1, 1)

        def body(i, carry):
            c0 = NBUF * i
            for j in range(NBUF):
                c = c0 + j
                wait_gather(c, j)
                scatter(c, j)
                bn = (j + 2) % NBUF

                def wait_prev(cc=c - 2, bb=bn):
                    wait_scatter(cc, bb)

                def refill(cc=c + 2, bb=bn):
                    gather(cc, bb)

                pl.when(c >= 2)(wait_prev)
                pl.when(c + 2 < CPW)(refill)

            def restage():
                half = lax.rem(c0 // 16 + 1, 2) * 16
                off = pl.multiple_of(ebase + c0 + 16, 8)
                pltpu.sync_copy(src_hbm.at[pl.ds(off, 16)],
                                sidx.at[pl.ds(half, 16)])
                pltpu.sync_copy(dst_hbm.at[pl.ds(off, 16)],
                                didx.at[pl.ds(half, 16)])

            pl.when(jnp.logical_and(lax.rem(c0, 16) == 0, c0 + 16 < CPW))(restage)
            return carry

        lax.fori_loop(0, CPW // NBUF, body, 0)
        wait_scatter(CPW - 2, (CPW - 2) % NBUF)
        wait_scatter(CPW - 1, (CPW - 1) % NBUF)

        plsc.subcore_barrier()
        pltpu.sync_copy(acc.at[pl.ds(r0, RPT)], out_hbm.at[cid].at[pl.ds(r0, RPT)])

    return _spmm


_spmm_f = _make_spmm(F)

RB = 400
NRB = N // RB


def _mm_body(x_ref, w_ref, o_ref):
    o_ref[...] = jnp.dot(x_ref[...], w_ref[...], preferred_element_type=jnp.float32)


def _matmul(x, w):
    n, kk = x.shape
    fo = w.shape[1]
    return pl.pallas_call(
        _mm_body,
        grid=(NRB,),
        in_specs=[pl.BlockSpec((RB, kk), lambda i: (i, 0)),
                  pl.BlockSpec((kk, fo), lambda i: (0, 0))],
        out_specs=pl.BlockSpec((RB, fo), lambda i: (i, 0)),
        out_shape=jax.ShapeDtypeStruct((n, fo), jnp.float32),
    )(x, w)


def _pool_body(b_ref, h_ref, s_ref, c_ref):
    g = b_ref[0, 0, :]
    oh = (lax.broadcasted_iota(jnp.int32, (NGR, RB), 0) == g[None, :]).astype(jnp.float32)
    s = jnp.dot(oh, h_ref[...], preferred_element_type=jnp.float32)
    cc = jnp.broadcast_to(jnp.sum(oh, axis=1, keepdims=True), (NGR, 128))

    @pl.when(pl.program_id(0) == 0)
    def _():
        s_ref[...] = jnp.zeros_like(s_ref)
        c_ref[...] = jnp.zeros_like(c_ref)

    s_ref[...] += s
    c_ref[...] += cc


def _pool(batch3, h2):
    return pl.pallas_call(
        _pool_body,
        grid=(NRB,),
        in_specs=[pl.BlockSpec((1, 1, RB), lambda i: (i, 0, 0)),
                  pl.BlockSpec((RB, F_OUT), lambda i: (i, 0))],
        out_specs=[pl.BlockSpec((NGR, F_OUT), lambda i: (0, 0)),
                   pl.BlockSpec((NGR, 128), lambda i: (0, 0))],
        out_shape=[jax.ShapeDtypeStruct((NGR, F_OUT), jnp.float32),
                   jax.ShapeDtypeStruct((NGR, 128), jnp.float32)],
    )(batch3, h2)


def _pad_table(t):
    return jnp.pad(t, ((0, NPAD - N), (0, 0)))


@jax.jit
def kernel(x, edge_index, batch, W1a, b1a, W1b, b1b, W2a, b2a, W2b, b2b):
    row = edge_index[0].astype(jnp.int32)
    col = edge_index[1].astype(jnp.int32)
    pad = (jnp.arange(EPAD - E, dtype=jnp.int32) % 16) + N
    colp = jnp.concatenate([col, pad])
    rowp = jnp.concatenate([row, pad])
    srcS = jnp.concatenate([colp, rowp]).reshape(2 * NS * CPW, CHUNK)
    dstS = jnp.concatenate([rowp, colp]).reshape(2 * NS * CPW, CHUNK)
    zero = jnp.zeros((NPAD, F), jnp.float32)

    def propagate2(u2):
        return _spmm_f(u2, srcS, dstS, zero)[:, :N]

    ones_t = jnp.ones((2, NPAD, F), jnp.float32)
    deg = propagate2(ones_t)[0, :, 0]
    dis = jnp.where(deg > 0, lax.rsqrt(jnp.maximum(deg, 1e-12)), 0.0)
    disc = dis[:, None]
    dis2c = (dis * dis)[:, None]
    sqdegc = jnp.sqrt(deg)[:, None]
    mask0 = (deg == 0.0).astype(jnp.float32)[:, None]
    pk = [1.0, -1.0 / 3.0]
    for _ in range(2, K):
        pk.append(-(2.0 / 3.0) * pk[-1] - pk[-2])

    def layer(v, Wa, ba, Wb, bb):
        W2 = (Wa, Wb)
        u0 = disc * v
        us = [(u0, u0)]
        u2 = jnp.stack([_pad_table(u0)] * 2)
        outs = [_matmul(u0, W2[d][0]) for d in range(2)]
        for k in range(1, K):
            agg2 = propagate2(u2)
            nxt = []
            for d in range(2):
                pv = dis2c * agg2[d]
                if k == 1:
                    u = -us[0][d] / 3.0 - (2.0 / 3.0) * pv
                else:
                    u = -(2.0 / 3.0) * us[-1][d] - (4.0 / 3.0) * pv - us[-2][d]
                nxt.append(u)
                outs[d] = outs[d] + _matmul(u, W2[d][k])
            us.append(tuple(nxt))
            if k < K - 1:
                u2 = jnp.stack([_pad_table(nxt[0]), _pad_table(nxt[1])])
        Wp = [sum(pk[k] * W2[d][k] for k in range(K)) for d in range(2)]
        deg0 = [_matmul(v, Wp[d]) for d in range(2)]
        oa = sqdegc * outs[0] + mask0 * deg0[0] + ba
        ob = sqdegc * outs[1] + mask0 * deg0[1] + bb
        return jax.nn.relu(jnp.concatenate([oa, ob], axis=1))

    h = layer(x, W1a, b1a, W1b, b1b)
    h2 = layer(h, W2a, b2a, W2b, b2b)

    batch3 = batch.astype(jnp.int32).reshape(NRB, 1, RB)
    sums, cnts = _pool(batch3, h2)
    return sums / jnp.maximum(cnts[:, :1], 1.0)


# speedup vs baseline: 1.0583x; 1.0583x over previous
import functools

import jax
import jax.numpy as jnp
from jax import lax
from jax.experimental import pallas as pl
from jax.experimental.pallas import tpu as pltpu
from jax.experimental.pallas import tpu_sc as plsc

N = 10000
E = 320000
F = 128
F_OUT = 512
NGR = 64
K = 5

NC, NS = 2, 16
NW = NC * NS
CHUNK = 80
CPW = 256
EPW = CHUNK * CPW
EPAD = EPW * NS
NPAD = 10112
RPT = NPAD // NS

_mesh = plsc.VectorSubcoreMesh(core_axis_name="c", subcore_axis_name="s")
NBUF = 4


def _make_spmm(width):
    @functools.partial(
        pl.kernel,
        out_type=jax.ShapeDtypeStruct((NC, NPAD, width), jnp.float32),
        mesh=_mesh,
        scratch_types=[
            pltpu.VMEM((32, CHUNK), jnp.int32),
            pltpu.VMEM((32, CHUNK), jnp.int32),
            pltpu.VMEM_SHARED((NPAD, width), jnp.float32),
        ] + [pltpu.VMEM((CHUNK, width), jnp.float32) for _ in range(NBUF)]
          + [pltpu.SemaphoreType.DMA for _ in range(NBUF + 1)],
    )
    def _spmm(u_hbm, src_hbm, dst_hbm, zero_hbm, out_hbm, sidx, didx, acc, *bufsems):
        rows = bufsems[:NBUF]
        sems = bufsems[NBUF:2 * NBUF]
        isem = bufsems[2 * NBUF]
        cid = lax.axis_index("c")
        sid = lax.axis_index("s")
        ebase = (cid * NS + sid) * CPW
        r0 = sid * RPT
        pltpu.sync_copy(zero_hbm.at[pl.ds(r0, RPT)], acc.at[pl.ds(r0, RPT)])
        pltpu.sync_copy(src_hbm.at[pl.ds(ebase, 32)], sidx)
        pltpu.sync_copy(dst_hbm.at[pl.ds(ebase, 32)], didx)
        plsc.subcore_barrier()

        uplane = u_hbm.at[cid]

        def gather(c, b):
            pltpu.async_copy(uplane.at[sidx.at[lax.rem(c, 32)]], rows[b], sems[b])

        def wait_gather(c, b):
            pltpu.make_async_copy(uplane.at[sidx.at[lax.rem(c, 32)]],
                                  rows[b], sems[b]).wait()

        def scatter(c, b):
            pltpu.async_copy(rows[b], acc.at[didx.at[lax.rem(c, 32)]],
                             sems[b], add=True)

        def wait_scatter(c, b):
            pltpu.make_async_copy(rows[b], acc.at[didx.at[lax.rem(c, 32)]],
                                  sems[b]).wait()

        gather(0, 0)
        gather(1, 1)

        def idx_copies(blk):
            half = lax.rem(blk, 2) * 16
            off = pl.multiple_of(ebase + blk * 16, 8)
            return (pltpu.make_async_copy(src_hbm.at[pl.ds(off, 16)],
                                          sidx.at[pl.ds(half, 16)], isem),
                    pltpu.make_async_copy(dst_hbm.at[pl.ds(off, 16)],
                                          didx.at[pl.ds(half, 16)], isem))

        def body(i, carry):
            c0 = NBUF * i

            def idx_wait():
                for cp in idx_copies((c0 - 8) // 16 + 1):
                    cp.wait()

            pl.when(jnp.logical_and(lax.rem(c0, 16) == 8, c0 + 8 < CPW))(idx_wait)

            for j in range(NBUF):
                c = c0 + j
                wait_gather(c, j)
                scatter(c, j)
                bn = (j + 2) % NBUF

                def wait_prev(cc=c - 2, bb=bn):
                    wait_scatter(cc, bb)

                def refill(cc=c + 2, bb=bn):
                    gather(cc, bb)

                pl.when(c >= 2)(wait_prev)
                pl.when(c + 2 < CPW)(refill)

            def restage():
                for cp in idx_copies(c0 // 16 + 1):
                    cp.start()

            pl.when(jnp.logical_and(lax.rem(c0, 16) == 0, c0 + 16 < CPW))(restage)
            return carry

        lax.fori_loop(0, CPW // NBUF, body, 0)
        wait_scatter(CPW - 2, (CPW - 2) % NBUF)
        wait_scatter(CPW - 1, (CPW - 1) % NBUF)

        plsc.subcore_barrier()
        pltpu.sync_copy(acc.at[pl.ds(r0, RPT)], out_hbm.at[cid].at[pl.ds(r0, RPT)])

    return _spmm


_spmm_f = _make_spmm(F)

RB = 400
NRB = N // RB


def _mm_body(x_ref, w_ref, o_ref):
    o_ref[...] = jnp.dot(x_ref[...], w_ref[...], preferred_element_type=jnp.float32)


def _matmul(x, w):
    n, kk = x.shape
    fo = w.shape[1]
    return pl.pallas_call(
        _mm_body,
        grid=(NRB,),
        in_specs=[pl.BlockSpec((RB, kk), lambda i: (i, 0)),
                  pl.BlockSpec((kk, fo), lambda i: (0, 0))],
        out_specs=pl.BlockSpec((RB, fo), lambda i: (i, 0)),
        out_shape=jax.ShapeDtypeStruct((n, fo), jnp.float32),
    )(x, w)


def _pool_body(b_ref, h_ref, s_ref, c_ref):
    g = b_ref[0, 0, :]
    oh = (lax.broadcasted_iota(jnp.int32, (NGR, RB), 0) == g[None, :]).astype(jnp.float32)
    s = jnp.dot(oh, h_ref[...], preferred_element_type=jnp.float32)
    cc = jnp.broadcast_to(jnp.sum(oh, axis=1, keepdims=True), (NGR, 128))

    @pl.when(pl.program_id(0) == 0)
    def _():
        s_ref[...] = jnp.zeros_like(s_ref)
        c_ref[...] = jnp.zeros_like(c_ref)

    s_ref[...] += s
    c_ref[...] += cc


def _pool(batch3, h2):
    return pl.pallas_call(
        _pool_body,
        grid=(NRB,),
        in_specs=[pl.BlockSpec((1, 1, RB), lambda i: (i, 0, 0)),
                  pl.BlockSpec((RB, F_OUT), lambda i: (i, 0))],
        out_specs=[pl.BlockSpec((NGR, F_OUT), lambda i: (0, 0)),
                   pl.BlockSpec((NGR, 128), lambda i: (0, 0))],
        out_shape=[jax.ShapeDtypeStruct((NGR, F_OUT), jnp.float32),
                   jax.ShapeDtypeStruct((NGR, 128), jnp.float32)],
    )(batch3, h2)


def _pad_table(t):
    return jnp.pad(t, ((0, NPAD - N), (0, 0)))


@jax.jit
def kernel(x, edge_index, batch, W1a, b1a, W1b, b1b, W2a, b2a, W2b, b2b):
    row = edge_index[0].astype(jnp.int32)
    col = edge_index[1].astype(jnp.int32)
    pad = (jnp.arange(EPAD - E, dtype=jnp.int32) % 16) + N
    colp = jnp.concatenate([col, pad])
    rowp = jnp.concatenate([row, pad])
    srcS = jnp.concatenate([colp, rowp]).reshape(2 * NS * CPW, CHUNK)
    dstS = jnp.concatenate([rowp, colp]).reshape(2 * NS * CPW, CHUNK)
    zero = jnp.zeros((NPAD, F), jnp.float32)

    def propagate2(u2):
        return _spmm_f(u2, srcS, dstS, zero)[:, :N]

    ones_t = jnp.ones((2, NPAD, F), jnp.float32)
    deg = propagate2(ones_t)[0, :, 0]
    dis = jnp.where(deg > 0, lax.rsqrt(jnp.maximum(deg, 1e-12)), 0.0)
    disc = dis[:, None]
    def layer(v, Wa, ba, Wb, bb):
        W2 = (Wa, Wb)
        txs = [(v, v)]
        u0 = _pad_table(disc * v)
        u2 = jnp.stack([u0, u0])
        outs = [_matmul(v, W2[d][0]) for d in range(2)]
        for k in range(1, K):
            agg2 = propagate2(u2)
            nxt = []
            for d in range(2):
                pv = disc * agg2[d]
                if k == 1:
                    tx = -txs[0][d] / 3.0 - (2.0 / 3.0) * pv
                else:
                    tx = -(2.0 / 3.0) * txs[-1][d] - (4.0 / 3.0) * pv - txs[-2][d]
                nxt.append(tx)
                outs[d] = outs[d] + _matmul(tx, W2[d][k])
            txs.append(tuple(nxt))
            if k < K - 1:
                u2 = jnp.stack([_pad_table(disc * nxt[0]), _pad_table(disc * nxt[1])])
        return jax.nn.relu(jnp.concatenate([outs[0] + ba, outs[1] + bb], axis=1))

    h = layer(x, W1a, b1a, W1b, b1b)
    h2 = layer(h, W2a, b2a, W2b, b2b)

    batch3 = batch.astype(jnp.int32).reshape(NRB, 1, RB)
    sums, cnts = _pool(batch3, h2)
    return sums / jnp.maximum(cnts[:, :1], 1.0)
